# baseline (speedup 1.0000x reference)
import functools

import jax
import jax.numpy as jnp
from jax import lax
from jax.experimental import pallas as pl
from jax.experimental.pallas import tpu as pltpu
from jax.experimental.pallas import tpu_sc as plsc

D_MODEL = 128
BATCH = 16384
_NUM_CORES = 2
_NUM_SUBCORES = 16
_NW = _NUM_CORES * _NUM_SUBCORES
_BPW = BATCH // _NW

_TABLE_ROWS = 1000
_K = 4

_mesh = plsc.VectorSubcoreMesh(core_axis_name="c", subcore_axis_name="s")


@functools.partial(
    pl.kernel,
    mesh=_mesh,
    out_type=jax.ShapeDtypeStruct((BATCH, D_MODEL), jnp.float32),
    scratch_types=[
        pltpu.VMEM((_BPW,), jnp.int32),
        pltpu.VMEM((_BPW, D_MODEL), jnp.float32),
        pltpu.VMEM_SHARED((_TABLE_ROWS, D_MODEL), jnp.float32),
        pltpu.SemaphoreType.DMA,
        pltpu.SemaphoreType.DMA,
        pltpu.SemaphoreType.DMA,
    ],
)
def _pe_gather(t_hbm, pe_hbm, out_hbm, idx_v, rows_v, table_s, sem, ssem, hsem):
    sid = lax.axis_index("s")
    wid = sid * _NUM_CORES + lax.axis_index("c")
    base = wid * _BPW
    pltpu.sync_copy(t_hbm.at[pl.ds(base, 32)], idx_v.at[pl.ds(0, 32)])
    bounds = [0, 32, 128, 256, 384, _BPW]
    gathers = [
        pltpu.async_copy(
            pe_hbm.at[idx_v.at[pl.ds(0, 32)]], rows_v.at[pl.ds(0, 32)], hsem
        )
    ]
    pltpu.sync_copy(
        t_hbm.at[pl.ds(base + 32, _BPW - 32)], idx_v.at[pl.ds(32, _BPW - 32)]
    )

    @pl.when(sid < 7)
    def _():
        pltpu.sync_copy(
            pe_hbm.at[pl.ds(sid * 128, 128)], table_s.at[pl.ds(sid * 128, 128)]
        )

    @pl.when(sid == 7)
    def _():
        pltpu.sync_copy(pe_hbm.at[pl.ds(896, 104)], table_s.at[pl.ds(896, 104)])

    plsc.subcore_barrier()
    gathers += [
        pltpu.async_copy(
            table_s.at[idx_v.at[pl.ds(bounds[i], bounds[i + 1] - bounds[i])]],
            rows_v.at[pl.ds(bounds[i], bounds[i + 1] - bounds[i])],
            sem,
        )
        for i in range(1, len(bounds) - 1)
    ]
    scatters = []
    for i in range(len(bounds) - 1):
        lo, n = bounds[i], bounds[i + 1] - bounds[i]
        gathers[i].wait()
        scatters.append(
            pltpu.async_copy(
                rows_v.at[pl.ds(lo, n)], out_hbm.at[pl.ds(base + lo, n)], ssem
            )
        )
    for s in scatters:
        s.wait()


def kernel(t, pe):
    return _pe_gather(t, pe)


# speedup vs baseline: 1.0177x; 1.0177x over previous
import functools

import jax
import jax.numpy as jnp
from jax import lax
from jax.experimental import pallas as pl
from jax.experimental.pallas import tpu as pltpu
from jax.experimental.pallas import tpu_sc as plsc

D_MODEL = 128
BATCH = 16384
_NUM_CORES = 2
_NUM_SUBCORES = 16
_NW = _NUM_CORES * _NUM_SUBCORES
_BPW = BATCH // _NW

_TABLE_ROWS = 1000
_K = 4

_mesh = plsc.VectorSubcoreMesh(core_axis_name="c", subcore_axis_name="s")


@functools.partial(
    pl.kernel,
    mesh=_mesh,
    out_type=jax.ShapeDtypeStruct((BATCH, D_MODEL), jnp.float32),
    scratch_types=[
        pltpu.VMEM((_BPW,), jnp.int32),
        pltpu.VMEM((_BPW, D_MODEL), jnp.float32),
        pltpu.VMEM_SHARED((_TABLE_ROWS, D_MODEL), jnp.float32),
        pltpu.SemaphoreType.DMA,
        pltpu.SemaphoreType.DMA,
    ],
)
def _pe_gather(t_hbm, pe_hbm, out_hbm, idx_v, rows_v, table_s, sem, ssem):
    sid = lax.axis_index("s")
    wid = sid * _NUM_CORES + lax.axis_index("c")
    base = wid * _BPW
    pltpu.sync_copy(t_hbm.at[pl.ds(base, _BPW)], idx_v)
    bounds = [0, 32, 128, 256, 384, _BPW]
    gathers = [
        pltpu.async_copy(
            pe_hbm.at[idx_v.at[pl.ds(0, 32)]], rows_v.at[pl.ds(0, 32)], ssem
        )
    ]

    @pl.when(sid < 7)
    def _():
        pltpu.sync_copy(
            pe_hbm.at[pl.ds(sid * 128, 128)], table_s.at[pl.ds(sid * 128, 128)]
        )

    @pl.when(sid == 7)
    def _():
        pltpu.sync_copy(pe_hbm.at[pl.ds(896, 104)], table_s.at[pl.ds(896, 104)])

    plsc.subcore_barrier()
    gathers += [
        pltpu.async_copy(
            table_s.at[idx_v.at[pl.ds(bounds[i], bounds[i + 1] - bounds[i])]],
            rows_v.at[pl.ds(bounds[i], bounds[i + 1] - bounds[i])],
            sem,
        )
        for i in range(1, len(bounds) - 1)
    ]
    scatters = []
    for i in range(len(bounds) - 1):
        lo, n = bounds[i], bounds[i + 1] - bounds[i]
        gathers[i].wait()
        scatters.append(
            pltpu.async_copy(
                rows_v.at[pl.ds(lo, n)], out_hbm.at[pl.ds(base + lo, n)], ssem
            )
        )
    for s in scatters:
        s.wait()


def kernel(t, pe):
    return _pe_gather(t, pe)
